# baseline (speedup 1.0000x reference)
import functools

import jax
import jax.numpy as jnp
from jax import lax
from jax.experimental import pallas as pl
from jax.experimental.pallas import tpu as pltpu
from jax.experimental.pallas import tpu_sc as plsc

_L = 16


@functools.lru_cache(maxsize=None)
def _make_slab_gather(T, B, D, Vp):
    info = plsc.get_sparse_core_info()
    nc, ns = info.num_cores, info.num_subcores
    nw = nc * ns
    assert B == 128 * nw and D == 64 and T % 2 == 0
    mesh = plsc.VectorSubcoreMesh(core_axis_name="c", subcore_axis_name="s")

    @functools.partial(
        pl.kernel,
        out_type=jax.ShapeDtypeStruct((T, D, B), jnp.float32),
        mesh=mesh,
        scratch_types=[
            pltpu.VMEM((T, 128), jnp.int32),
            [pltpu.VMEM((128,), jnp.int32) for _ in range(2)],
            [pltpu.VMEM((128,), jnp.int32) for _ in range(2)],
            [pltpu.VMEM((128, 2 * D + 1), jnp.float32) for _ in range(2)],
            [pltpu.VMEM((D, 128), jnp.float32) for _ in range(2)],
            [pltpu.SemaphoreType.DMA for _ in range(2)],
            [pltpu.SemaphoreType.DMA for _ in range(2)],
        ],
        compiler_params=pltpu.CompilerParams(
            needs_layout_passes=False, disable_bounds_checks=True),
    )
    def slab_kernel(tok_hbm, packed_hbm, out_hbm, itile, idx2, colbit, rows,
                    dst, gsems, ssems):
        wid = lax.axis_index("s") * nc + lax.axis_index("c")
        col0 = wid * 128
        iota = lax.iota(jnp.int32, _L)
        jvecs = [m * _L + iota for m in range(8)]

        pltpu.sync_copy(tok_hbm.at[:, pl.ds(col0, 128)], itile)

        def prep(s, p):
            for m in range(8):
                v = itile[s, pl.ds(m * _L, _L)]
                idx2[p][pl.ds(m * _L, _L)] = lax.shift_right_logical(v, 1)
                colbit[p][pl.ds(m * _L, _L)] = (v & 1) * D

        def g_start(p):
            pltpu.async_copy(
                packed_hbm.at[idx2[p]], rows[p].at[:, pl.ds(0, 2 * D)],
                gsems[p])

        def g_wait(p):
            pltpu.make_async_copy(
                packed_hbm.at[idx2[p]], rows[p].at[:, pl.ds(0, 2 * D)],
                gsems[p]).wait()

        def s_start(t, p):
            pltpu.async_copy(
                dst[p], out_hbm.at[t, :, pl.ds(col0, 128)], ssems[p])

        def s_wait(t, p):
            pltpu.make_async_copy(
                dst[p], out_hbm.at[t, :, pl.ds(col0, 128)], ssems[p]).wait()

        def transpose(p):
            cb = [colbit[p][pl.ds(m * _L, _L)] for m in range(8)]

            @plsc.parallel_loop(0, D, step=1, unroll=8)
            def _(f):
                for m in range(8):
                    got = plsc.load_gather(rows[p], [jvecs[m], cb[m] + f])
                    dst[p][f, pl.ds(m * _L, _L)] = got

        prep(0, 0)
        g_start(0)

        def body(o, carry):
            for par in (0, 1):
                s = 2 * o + par
                nxt = s + 1
                q = 1 - par

                @pl.when(nxt < T)
                def _():
                    prep(nxt, q)

                    @pl.when(nxt >= 2)
                    def _():
                        s_wait(nxt - 2, q)

                    g_start(q)

                g_wait(par)
                transpose(par)
                s_start(s, par)
            return carry

        lax.fori_loop(0, T // 2, body, 0)
        s_wait(T - 2, 0)
        s_wait(T - 1, 1)

    return slab_kernel


def kernel(tokens, embed_weights):
    b, t = tokens.shape
    v, d = embed_weights.shape
    tok_t = tokens.T
    packed = jnp.concatenate(
        [embed_weights[0::2], embed_weights[1::2]], axis=1)
    out3 = _make_slab_gather(t, b, d, v // 2)(tok_t, packed)
    return jnp.transpose(out3, (2, 0, 1))


# speedup vs baseline: 9.0310x; 9.0310x over previous
import functools

import jax
import jax.numpy as jnp
from jax import lax
from jax.experimental import pallas as pl
from jax.experimental.pallas import tpu as pltpu
from jax.experimental.pallas import tpu_sc as plsc

_L = 16


@functools.lru_cache(maxsize=None)
def _make_row_gather(T, B, D, Vp):
    info = plsc.get_sparse_core_info()
    nc, ns = info.num_cores, info.num_subcores
    nw = nc * ns
    assert B == 128 * nw and D == 64 and T % 8 == 0
    mesh = plsc.VectorSubcoreMesh(core_axis_name="c", subcore_axis_name="s")

    @functools.partial(
        pl.kernel,
        out_type=jax.ShapeDtypeStruct((B * T, D), jnp.float32),
        mesh=mesh,
        scratch_types=[
            pltpu.VMEM((T, 128), jnp.int32),
            [pltpu.VMEM((T,), jnp.int32) for _ in range(2)],
            [pltpu.VMEM((T,), jnp.int32) for _ in range(2)],
            [pltpu.VMEM((T, 2 * D), jnp.float32) for _ in range(2)],
            [pltpu.VMEM((T, D), jnp.float32) for _ in range(2)],
            [pltpu.SemaphoreType.DMA for _ in range(2)],
            [pltpu.SemaphoreType.DMA for _ in range(2)],
        ],
        compiler_params=pltpu.CompilerParams(
            needs_layout_passes=False, disable_bounds_checks=True),
    )
    def row_kernel(tok_hbm, packed_hbm, out_hbm, itile, idx2, colbit, rows,
                   dst, gsems, ssems):
        wid = lax.axis_index("s") * nc + lax.axis_index("c")
        col0 = wid * 128
        iota = lax.iota(jnp.int32, _L)

        pltpu.sync_copy(tok_hbm.at[:, pl.ds(col0, 128)], itile)

        starts = list(range(0, T - _L + 1, _L))
        if starts[-1] != T - _L:
            starts.append(T - _L)

        def prep(b, p):
            bvec = iota * 0 + b
            for c0 in starts:
                v = plsc.load_gather(itile, [c0 + iota, bvec])
                idx2[p][pl.ds(c0, _L)] = lax.shift_right_logical(v, 1)
                colbit[p][pl.ds(c0, _L)] = (v & 1) * D

        def g_start(p):
            pltpu.async_copy(packed_hbm.at[idx2[p]], rows[p], gsems[p])

        def g_wait(p):
            pltpu.make_async_copy(
                packed_hbm.at[idx2[p]], rows[p], gsems[p]).wait()

        def s_start(b, p):
            pltpu.async_copy(
                dst[p], out_hbm.at[pl.ds((col0 + b) * T, T), :], ssems[p])

        def s_wait(b, p):
            pltpu.make_async_copy(
                dst[p], out_hbm.at[pl.ds((col0 + b) * T, T), :],
                ssems[p]).wait()

        def extract(p):
            @plsc.parallel_loop(0, T, step=1, unroll=4)
            def _(j):
                jv = iota * 0 + j
                cb = plsc.load_gather(colbit[p], [jv])
                for m in range(D // _L):
                    got = plsc.load_gather(rows[p], [jv, cb + m * _L + iota])
                    dst[p][j, pl.ds(m * _L, _L)] = got

        prep(0, 0)
        g_start(0)

        def body(o, carry):
            for par in (0, 1):
                b = 2 * o + par
                nxt = b + 1
                q = 1 - par

                @pl.when(nxt < 128)
                def _():
                    prep(nxt, q)

                    @pl.when(nxt >= 2)
                    def _():
                        s_wait(nxt - 2, q)

                    g_start(q)

                g_wait(par)
                extract(par)
                s_start(b, par)
            return carry

        lax.fori_loop(0, 64, body, 0)
        s_wait(126, 0)
        s_wait(127, 1)

    return row_kernel


def kernel(tokens, embed_weights):
    b, t = tokens.shape
    v, d = embed_weights.shape
    tok_t = tokens.T
    packed = embed_weights.reshape(v // 2, 2 * d)
    out2d = _make_row_gather(t, b, d, v // 2)(tok_t, packed)
    return out2d.reshape(b, t, d)
